# baseline (speedup 1.0000x reference)
import functools

import jax
import jax.numpy as jnp
from jax import lax
from jax.experimental import pallas as pl
from jax.experimental.pallas import tpu as pltpu
from jax.experimental.pallas import tpu_sc as plsc

N = 4096
DIM = 3
H = 64
K = 32
OUT = 3

RB = 256
NBLK_TOPK = N // RB
NB = 512
NBLK_LAYER = N // NB
E = N * K


_S = 32
_G = N // _S
_RCAP = 10
_BUFT = _RCAP * _S + 2 * K


def _topk_body(xb_ref, xt_ref, we_ref, be_ref, wt_ref, wm_ref,
               dn_ref, idx_ref, h_ref, a_ref, b_ref, d_scr, vbuf, ibuf):
    blk = pl.program_id(0)
    xb = xb_ref[...]
    xt = xt_ref[...]
    d0 = xb[:, 0][:, None, None] - xt[0][None]
    d1 = xb[:, 1][:, None, None] - xt[1][None]
    d2 = xb[:, 2][:, None, None] - xt[2][None]
    dsq = d0 * d0 + d1 * d1 + d2 * d2

    rows = blk * RB + lax.broadcasted_iota(jnp.int32, (RB, _S, _G), 0)
    cols3 = (lax.broadcasted_iota(jnp.int32, (RB, _S, _G), 1) * _G
             + lax.broadcasted_iota(jnp.int32, (RB, _S, _G), 2))
    d_scr[...] = jnp.where(rows == cols3, jnp.inf, dsq)
    vbuf[...] = jnp.full((RB, _BUFT), jnp.inf, jnp.float32)
    ibuf[...] = jnp.zeros((RB, _BUFT), jnp.int32)

    seg_off = lax.broadcasted_iota(jnp.int32, (RB, _S), 1) * _G
    lanebuf = lax.broadcasted_iota(jnp.int32, (RB, _BUFT), 1)
    roundlane = lanebuf // _S

    def round_body(carry):
        r, _ = carry
        d3 = d_scr[...]
        jio = lax.broadcasted_iota(jnp.int32, (RB, _S, _G), 2)
        sm = jnp.min(d3, axis=2)
        sil = jnp.min(jnp.where(d3 <= sm[:, :, None], jio, _G), axis=2)
        d_scr[...] = jnp.where(jio == sil[:, :, None], jnp.inf, d3)
        sig = sil + seg_off
        bm = jnp.min(sm, axis=1)
        bi = jnp.min(jnp.where(sm <= bm[:, None], sig, N), axis=1)
        sm_t = jnp.broadcast_to(
            sm[:, None, :], (RB, _BUFT // _S, _S)).reshape(RB, _BUFT)
        si_t = jnp.broadcast_to(
            sig[:, None, :], (RB, _BUFT // _S, _S)).reshape(RB, _BUFT)
        sel = roundlane == r
        vbuf[...] = jnp.where(sel, sm_t, vbuf[...])
        ibuf[...] = jnp.where(sel, si_t, ibuf[...])
        bv, bix = vbuf[...], ibuf[...]
        le = (bv < bm[:, None]) | ((bv == bm[:, None])
                                   & (bix <= bi[:, None]))
        cnt = jnp.sum(le.astype(jnp.int32), axis=1)
        return r + 1, cnt

    def round_cond(carry):
        r, cnt = carry
        return jnp.logical_and(r < _RCAP, jnp.min(cnt) < K)

    _, cnt_fin = lax.while_loop(
        round_cond, round_body,
        (jnp.int32(0), jnp.zeros((RB,), jnp.int32)))

    @pl.when(jnp.min(cnt_fin) < K)
    def _fallback():
        def fb(t, c):
            d3 = d_scr[...]
            m = jnp.min(jnp.min(d3, axis=2), axis=1)
            ii = jnp.min(jnp.min(
                jnp.where(d3 <= m[:, None, None], cols3, N), axis=2), axis=1)
            d_scr[...] = jnp.where(cols3 == ii[:, None, None], jnp.inf, d3)
            sel = lanebuf == _RCAP * _S + t
            vbuf[...] = jnp.where(sel, m[:, None], vbuf[...])
            ibuf[...] = jnp.where(sel, ii[:, None], ibuf[...])
            return c
        lax.fori_loop(0, K, fb, 0)

    lane = lax.broadcasted_iota(jnp.int32, (RB, K), 1)

    def ext(t, carry):
        dn_acc, idx_acc = carry
        bv, bix = vbuf[...], ibuf[...]
        m = jnp.min(bv, axis=1)
        ii = jnp.min(jnp.where(bv <= m[:, None], bix, N), axis=1)
        vbuf[...] = jnp.where(
            (bv == m[:, None]) & (bix == ii[:, None]), jnp.inf, bv)
        dn_acc = jnp.where(lane == t,
                           jnp.sqrt(jnp.maximum(m, 0.0))[:, None], dn_acc)
        idx_acc = jnp.where(lane == t, ii[:, None], idx_acc)
        return dn_acc, idx_acc

    dn_acc, idx_acc = lax.fori_loop(
        0, K, ext,
        (jnp.zeros((RB, K), jnp.float32), jnp.zeros((RB, K), jnp.int32)))
    dn_ref[...] = dn_acc
    idx_ref[...] = idx_acc

    h = (xb[:, 0:1] * we_ref[0:1, :] + xb[:, 1:2] * we_ref[1:2, :]
         + xb[:, 2:3] * we_ref[2:3, :] + be_ref[...])
    h_ref[...] = h
    a_ref[...] = jnp.dot(h, wt_ref[...], preferred_element_type=jnp.float32)
    b_ref[...] = jnp.dot(h, wm_ref[...], preferred_element_type=jnp.float32)


def _topk_call(x, xt, W_embed, b_embed2, wt0, wm0):
    full = lambda shape: pl.BlockSpec(shape, lambda i: (0, 0))
    return pl.pallas_call(
        _topk_body,
        grid=(NBLK_TOPK,),
        in_specs=[
            pl.BlockSpec((RB, DIM), lambda i: (i, 0)),
            pl.BlockSpec((DIM, _S, _G), lambda i: (0, 0, 0)),
            full((DIM, H)),
            full((1, H)),
            full((H, H)),
            full((H, _GW)),
        ],
        out_specs=[
            pl.BlockSpec((RB, K), lambda i: (i, 0)),
            pl.BlockSpec((RB, K), lambda i: (i, 0)),
            pl.BlockSpec((RB, H), lambda i: (i, 0)),
            pl.BlockSpec((RB, H), lambda i: (i, 0)),
            pl.BlockSpec((RB, _GW), lambda i: (i, 0)),
        ],
        out_shape=[
            jax.ShapeDtypeStruct((N, K), jnp.float32),
            jax.ShapeDtypeStruct((N, K), jnp.int32),
            jax.ShapeDtypeStruct((N, H), jnp.float32),
            jax.ShapeDtypeStruct((N, H), jnp.float32),
            jax.ShapeDtypeStruct((N, _GW), jnp.float32),
        ],
        scratch_shapes=[pltpu.VMEM((RB, _S, _G), jnp.float32),
                        pltpu.VMEM((RB, _BUFT), jnp.float32),
                        pltpu.VMEM((RB, _BUFT), jnp.int32)],
        compiler_params=pltpu.CompilerParams(
            dimension_semantics=("parallel",)),
    )(x, xt, W_embed, b_embed2, wt0, wm0)


_SC_NW = 32
_PER_W = E // _SC_NW
_CHUNK = 512
_NCH = _PER_W // _CHUNK
_SUB = _CHUNK // 128
_GW = 2 * H


def _sc_gather_body(table_hbm, idx_hbm, out_hbm, idx_v, rows_v, sem):
    wid = lax.axis_index("s") * 2 + lax.axis_index("c")
    for ch in range(_NCH):
        row0 = pl.multiple_of(wid * _PER_W + ch * _CHUNK, _CHUNK)
        pltpu.sync_copy(idx_hbm.at[pl.ds(pl.multiple_of(row0 // 128, _SUB),
                                         _SUB)], idx_v)
        copies = []
        for j in range(_SUB):
            copies.append(pltpu.async_copy(
                table_hbm.at[idx_v.at[j]],
                rows_v.at[pl.ds(j * 128, 128)], sem))
        for c in copies:
            c.wait()
        pltpu.sync_copy(rows_v, out_hbm.at[pl.ds(row0, _CHUNK)])


@functools.lru_cache(maxsize=1)
def _sc_gather():
    return functools.partial(
        pl.kernel,
        out_type=jax.ShapeDtypeStruct((E, _GW), jnp.float32),
        mesh=plsc.VectorSubcoreMesh(core_axis_name="c", subcore_axis_name="s",
                                    num_cores=2, num_subcores=16),
        scratch_types=[
            pltpu.VMEM((_SUB, 128), jnp.int32),
            pltpu.VMEM((_CHUNK, _GW), jnp.float32),
            pltpu.SemaphoreType.DMA,
        ],
    )(_sc_gather_body)


def _silu(x):
    return x * (1.0 / (1.0 + jnp.exp(-x)))


def _layer_body(is_last,
                h_ref, a_ref, g_ref, dn_ref, wd_ref, bmi_ref, wmo_ref, bmo_ref,
                wuh_ref, wua_ref, bui_ref, wuo_ref, buo_ref, wn1_ref, wn2_ref,
                o1_ref, o2_ref, o3_ref):
    h = h_ref[...]
    g = g_ref[:, :H].reshape(NB, K, H)
    pre = (g + a_ref[...][:, None, :]
           + dn_ref[...][:, :, None] * wd_ref[...][None, :, :]
           + bmi_ref[...][None, :, :])
    s = jnp.sum(_silu(pre), axis=1)
    agg = (jnp.dot(s, wmo_ref[...], preferred_element_type=jnp.float32)
           + float(K) * bmo_ref[...])
    ui = (jnp.dot(h, wuh_ref[...], preferred_element_type=jnp.float32)
          + jnp.dot(agg, wua_ref[...], preferred_element_type=jnp.float32)
          + bui_ref[...])
    upd = (jnp.dot(_silu(ui), wuo_ref[...], preferred_element_type=jnp.float32)
           + buo_ref[...])
    hn = h + upd
    if is_last:
        o1_ref[...] = (jnp.dot(hn, wn1_ref[...],
                               preferred_element_type=jnp.float32)
                       + wn2_ref[...])
    else:
        o1_ref[...] = hn
        o2_ref[...] = jnp.dot(hn, wn1_ref[...],
                              preferred_element_type=jnp.float32)
        o3_ref[...] = jnp.dot(hn, wn2_ref[...],
                              preferred_element_type=jnp.float32)


def _layer_call(is_last, h, A, G, dn, wd, bmi, wmo, bmo,
                wuh, wua, bui, wuo, buo, wn1, wn2):
    full = lambda shape: pl.BlockSpec(shape, lambda i: (0, 0))
    row = lambda w: pl.BlockSpec((NB, w), lambda i: (i, 0))
    if is_last:
        out_specs = [pl.BlockSpec((NB, OUT), lambda i: (i, 0))]
        out_shape = [jax.ShapeDtypeStruct((N, OUT), jnp.float32)]
        body = functools.partial(_layer_body, True)

        def wrapped(*refs):
            body(*refs, None, None)
    else:
        out_specs = [row(H), row(H), row(_GW)]
        out_shape = [jax.ShapeDtypeStruct((N, H), jnp.float32)] * 2 + [
            jax.ShapeDtypeStruct((N, _GW), jnp.float32)]
        wrapped = functools.partial(_layer_body, False)
    res = pl.pallas_call(
        wrapped,
        grid=(NBLK_LAYER,),
        in_specs=[
            row(H), row(H),
            pl.BlockSpec((NB * K, _GW), lambda i: (i, 0)),
            row(K),
            full((1, H)), full((1, H)), full((H, H)), full((1, H)),
            full((H, H)), full((H, H)), full((1, H)), full((H, H)),
            full((1, H)),
            full(wn1.shape), full(wn2.shape),
        ],
        out_specs=out_specs,
        out_shape=out_shape,
        compiler_params=pltpu.CompilerParams(
            dimension_semantics=("parallel",)),
    )(h, A, G, dn, wd, bmi, wmo, bmo, wuh, wua, bui, wuo, buo, wn1, wn2)
    return res


def kernel(x, W_embed, b_embed,
           W_msg_in_0, b_msg_in_0, W_msg_out_0, b_msg_out_0,
           W_upd_in_0, b_upd_in_0, W_upd_out_0, b_upd_out_0,
           W_msg_in_1, b_msg_in_1, W_msg_out_1, b_msg_out_1,
           W_upd_in_1, b_upd_in_1, W_upd_out_1, b_upd_out_1,
           W_msg_in_2, b_msg_in_2, W_msg_out_2, b_msg_out_2,
           W_upd_in_2, b_upd_in_2, W_upd_out_2, b_upd_out_2,
           W_out, b_out):
    msg_in = [W_msg_in_0, W_msg_in_1, W_msg_in_2]
    b_msg_in = [b_msg_in_0, b_msg_in_1, b_msg_in_2]
    msg_out = [W_msg_out_0, W_msg_out_1, W_msg_out_2]
    b_msg_out = [b_msg_out_0, b_msg_out_1, b_msg_out_2]
    upd_in = [W_upd_in_0, W_upd_in_1, W_upd_in_2]
    b_upd_in = [b_upd_in_0, b_upd_in_1, b_upd_in_2]
    upd_out = [W_upd_out_0, W_upd_out_1, W_upd_out_2]
    b_upd_out = [b_upd_out_0, b_upd_out_1, b_upd_out_2]

    r2 = lambda v: v.reshape(1, -1)
    padw = lambda w: jnp.pad(w, ((0, 0), (0, _GW - w.shape[1])))
    xt = x.T.reshape(DIM, _S, _G)
    dn, idx, h, A, B = _topk_call(
        x, xt, W_embed, r2(b_embed), W_msg_in_0[:H],
        padw(W_msg_in_0[H:2 * H]))
    idx2d = idx.reshape(E // 128, 128)

    out = None
    for l in range(3):
        G = _sc_gather()(B, idx2d)
        is_last = l == 2
        if is_last:
            wn1, wn2 = W_out, r2(b_out)
        else:
            wn1, wn2 = msg_in[l + 1][:H], padw(msg_in[l + 1][H:2 * H])
        res = _layer_call(
            is_last, h, A, G, dn,
            r2(msg_in[l][2 * H]), r2(b_msg_in[l]), msg_out[l],
            r2(b_msg_out[l]),
            upd_in[l][:H], upd_in[l][H:], r2(b_upd_in[l]),
            upd_out[l], r2(b_upd_out[l]), wn1, wn2)
        if is_last:
            out = res[0]
        else:
            h, A, B = res
    return out


# speedup vs baseline: 1.3644x; 1.3644x over previous
import functools

import jax
import jax.numpy as jnp
from jax import lax
from jax.experimental import pallas as pl
from jax.experimental.pallas import tpu as pltpu
from jax.experimental.pallas import tpu_sc as plsc

N = 4096
DIM = 3
H = 64
K = 32
OUT = 3

RB = 256
NBLK_TOPK = N // RB
NB = 512
NBLK_LAYER = N // NB
E = N * K


def _topk_body(xb_ref, xt_ref, we_ref, be_ref, wt_ref, wm_ref,
               dn_ref, idx_ref, h_ref, a_ref, b_ref, d_scr):
    blk = pl.program_id(0)
    xb = xb_ref[...]
    xt = xt_ref[...]
    d0 = xb[:, 0:1] - xt[0:1, :]
    d1 = xb[:, 1:2] - xt[1:2, :]
    d2 = xb[:, 2:3] - xt[2:3, :]
    dsq = d0 * d0 + d1 * d1 + d2 * d2

    rows = blk * RB + lax.broadcasted_iota(jnp.int32, (RB, N), 0)
    cols = lax.broadcasted_iota(jnp.int32, (RB, N), 1)
    d_scr[...] = jnp.where(rows == cols, jnp.inf, dsq)

    lane = lax.broadcasted_iota(jnp.int32, (RB, K), 1)

    def step(t, carry):
        dn_acc, idx_acc = carry
        d = d_scr[...]
        m = jnp.min(d, axis=1)
        idxi = jnp.min(jnp.where(d <= m[:, None], cols, N), axis=1)
        d_scr[...] = jnp.where(cols == idxi[:, None], jnp.inf, d)
        dn_acc = jnp.where(lane == t,
                           jnp.sqrt(jnp.maximum(m, 0.0))[:, None], dn_acc)
        idx_acc = jnp.where(lane == t, idxi[:, None], idx_acc)
        return dn_acc, idx_acc

    dn_acc, idx_acc = lax.fori_loop(
        0, K, step,
        (jnp.zeros((RB, K), jnp.float32), jnp.zeros((RB, K), jnp.int32)))
    dn_ref[...] = dn_acc
    idx_ref[...] = idx_acc

    h = (xb[:, 0:1] * we_ref[0:1, :] + xb[:, 1:2] * we_ref[1:2, :]
         + xb[:, 2:3] * we_ref[2:3, :] + be_ref[...])
    h_ref[...] = h
    a_ref[...] = jnp.dot(h, wt_ref[...], preferred_element_type=jnp.float32)
    b_ref[...] = jnp.dot(h, wm_ref[...], preferred_element_type=jnp.float32)


def _topk_call(x, xt, W_embed, b_embed2, wt0, wm0):
    full = lambda shape: pl.BlockSpec(shape, lambda i: (0, 0))
    return pl.pallas_call(
        _topk_body,
        grid=(NBLK_TOPK,),
        in_specs=[
            pl.BlockSpec((RB, DIM), lambda i: (i, 0)),
            full((DIM, N)),
            full((DIM, H)),
            full((1, H)),
            full((H, H)),
            full((H, _GW)),
        ],
        out_specs=[
            pl.BlockSpec((RB, K), lambda i: (i, 0)),
            pl.BlockSpec((RB, K), lambda i: (i, 0)),
            pl.BlockSpec((RB, H), lambda i: (i, 0)),
            pl.BlockSpec((RB, H), lambda i: (i, 0)),
            pl.BlockSpec((RB, _GW), lambda i: (i, 0)),
        ],
        out_shape=[
            jax.ShapeDtypeStruct((N, K), jnp.float32),
            jax.ShapeDtypeStruct((N, K), jnp.int32),
            jax.ShapeDtypeStruct((N, H), jnp.float32),
            jax.ShapeDtypeStruct((N, H), jnp.float32),
            jax.ShapeDtypeStruct((N, _GW), jnp.float32),
        ],
        scratch_shapes=[pltpu.VMEM((RB, N), jnp.float32)],
        compiler_params=pltpu.CompilerParams(
            dimension_semantics=("parallel",)),
    )(x, xt, W_embed, b_embed2, wt0, wm0)


_SC_NW = 32
_PER_W = E // _SC_NW
_CHUNK = 256
_NCH = _PER_W // _CHUNK
_SUB = _CHUNK // 128
_IDXR = _PER_W // 128
_GW = 2 * H


def _sc_gather_body(table_hbm, idx_hbm, out_hbm, idx_v, rows_v,
                    gsem0, gsem1, osem0, osem1):
    wid = lax.axis_index("s") * 2 + lax.axis_index("c")
    gsem = (gsem0, gsem1)
    osem = (osem0, osem1)
    pltpu.sync_copy(
        idx_hbm.at[pl.ds(pl.multiple_of(wid * _IDXR, _IDXR), _IDXR)], idx_v)
    outc = [None] * _NCH
    for ch in range(_NCH):
        p = ch & 1
        if ch >= 2:
            outc[ch - 2].wait()
        gc = []
        for j in range(_SUB):
            gc.append(pltpu.async_copy(
                table_hbm.at[idx_v.at[ch * _SUB + j]],
                rows_v.at[p, pl.ds(j * 128, 128)], gsem[p]))
        for c in gc:
            c.wait()
        row0 = pl.multiple_of(wid * _PER_W + ch * _CHUNK, _CHUNK)
        outc[ch] = pltpu.async_copy(
            rows_v.at[p], out_hbm.at[pl.ds(row0, _CHUNK)], osem[p])
    outc[_NCH - 2].wait()
    outc[_NCH - 1].wait()


@functools.lru_cache(maxsize=1)
def _sc_gather():
    return functools.partial(
        pl.kernel,
        out_type=jax.ShapeDtypeStruct((E, _GW), jnp.float32),
        mesh=plsc.VectorSubcoreMesh(core_axis_name="c", subcore_axis_name="s",
                                    num_cores=2, num_subcores=16),
        scratch_types=[
            pltpu.VMEM((_IDXR, 128), jnp.int32),
            pltpu.VMEM((2, _CHUNK, _GW), jnp.float32),
            pltpu.SemaphoreType.DMA,
            pltpu.SemaphoreType.DMA,
            pltpu.SemaphoreType.DMA,
            pltpu.SemaphoreType.DMA,
        ],
    )(_sc_gather_body)


def _silu(x):
    return x * (1.0 / (1.0 + jnp.exp(-x)))


def _layer_body(is_last,
                h_ref, a_ref, g_ref, dn_ref, wd_ref, bmi_ref, wmo_ref, bmo_ref,
                wuh_ref, wua_ref, bui_ref, wuo_ref, buo_ref, wn1_ref, wn2_ref,
                o1_ref, o2_ref, o3_ref):
    h = h_ref[...]
    g = g_ref[:, :H].reshape(NB, K, H)
    pre = (g + a_ref[...][:, None, :]
           + dn_ref[...][:, :, None] * wd_ref[...][None, :, :]
           + bmi_ref[...][None, :, :])
    s = jnp.sum(_silu(pre), axis=1)
    agg = (jnp.dot(s, wmo_ref[...], preferred_element_type=jnp.float32)
           + float(K) * bmo_ref[...])
    ui = (jnp.dot(h, wuh_ref[...], preferred_element_type=jnp.float32)
          + jnp.dot(agg, wua_ref[...], preferred_element_type=jnp.float32)
          + bui_ref[...])
    upd = (jnp.dot(_silu(ui), wuo_ref[...], preferred_element_type=jnp.float32)
           + buo_ref[...])
    hn = h + upd
    if is_last:
        o1_ref[...] = (jnp.dot(hn, wn1_ref[...],
                               preferred_element_type=jnp.float32)
                       + wn2_ref[...])
    else:
        o1_ref[...] = hn
        o2_ref[...] = jnp.dot(hn, wn1_ref[...],
                              preferred_element_type=jnp.float32)
        o3_ref[...] = jnp.dot(hn, wn2_ref[...],
                              preferred_element_type=jnp.float32)


def _layer_call(is_last, h, A, G, dn, wd, bmi, wmo, bmo,
                wuh, wua, bui, wuo, buo, wn1, wn2):
    full = lambda shape: pl.BlockSpec(shape, lambda i: (0, 0))
    row = lambda w: pl.BlockSpec((NB, w), lambda i: (i, 0))
    if is_last:
        out_specs = [pl.BlockSpec((NB, OUT), lambda i: (i, 0))]
        out_shape = [jax.ShapeDtypeStruct((N, OUT), jnp.float32)]
        body = functools.partial(_layer_body, True)

        def wrapped(*refs):
            body(*refs, None, None)
    else:
        out_specs = [row(H), row(H), row(_GW)]
        out_shape = [jax.ShapeDtypeStruct((N, H), jnp.float32)] * 2 + [
            jax.ShapeDtypeStruct((N, _GW), jnp.float32)]
        wrapped = functools.partial(_layer_body, False)
    res = pl.pallas_call(
        wrapped,
        grid=(NBLK_LAYER,),
        in_specs=[
            row(H), row(H),
            pl.BlockSpec((NB * K, _GW), lambda i: (i, 0)),
            row(K),
            full((1, H)), full((1, H)), full((H, H)), full((1, H)),
            full((H, H)), full((H, H)), full((1, H)), full((H, H)),
            full((1, H)),
            full(wn1.shape), full(wn2.shape),
        ],
        out_specs=out_specs,
        out_shape=out_shape,
        compiler_params=pltpu.CompilerParams(
            dimension_semantics=("parallel",)),
    )(h, A, G, dn, wd, bmi, wmo, bmo, wuh, wua, bui, wuo, buo, wn1, wn2)
    return res


def kernel(x, W_embed, b_embed,
           W_msg_in_0, b_msg_in_0, W_msg_out_0, b_msg_out_0,
           W_upd_in_0, b_upd_in_0, W_upd_out_0, b_upd_out_0,
           W_msg_in_1, b_msg_in_1, W_msg_out_1, b_msg_out_1,
           W_upd_in_1, b_upd_in_1, W_upd_out_1, b_upd_out_1,
           W_msg_in_2, b_msg_in_2, W_msg_out_2, b_msg_out_2,
           W_upd_in_2, b_upd_in_2, W_upd_out_2, b_upd_out_2,
           W_out, b_out):
    msg_in = [W_msg_in_0, W_msg_in_1, W_msg_in_2]
    b_msg_in = [b_msg_in_0, b_msg_in_1, b_msg_in_2]
    msg_out = [W_msg_out_0, W_msg_out_1, W_msg_out_2]
    b_msg_out = [b_msg_out_0, b_msg_out_1, b_msg_out_2]
    upd_in = [W_upd_in_0, W_upd_in_1, W_upd_in_2]
    b_upd_in = [b_upd_in_0, b_upd_in_1, b_upd_in_2]
    upd_out = [W_upd_out_0, W_upd_out_1, W_upd_out_2]
    b_upd_out = [b_upd_out_0, b_upd_out_1, b_upd_out_2]

    r2 = lambda v: v.reshape(1, -1)
    padw = lambda w: jnp.pad(w, ((0, 0), (0, _GW - w.shape[1])))
    xt = x.T
    dn, idx, h, A, B = _topk_call(
        x, xt, W_embed, r2(b_embed), W_msg_in_0[:H],
        padw(W_msg_in_0[H:2 * H]))
    idx2d = idx.reshape(E // 128, 128)

    out = None
    for l in range(3):
        G = _sc_gather()(B, idx2d)
        is_last = l == 2
        if is_last:
            wn1, wn2 = W_out, r2(b_out)
        else:
            wn1, wn2 = msg_in[l + 1][:H], padw(msg_in[l + 1][H:2 * H])
        res = _layer_call(
            is_last, h, A, G, dn,
            r2(msg_in[l][2 * H]), r2(b_msg_in[l]), msg_out[l],
            r2(b_msg_out[l]),
            upd_in[l][:H], upd_in[l][H:], r2(b_upd_in[l]),
            upd_out[l], r2(b_upd_out[l]), wn1, wn2)
        if is_last:
            out = res[0]
        else:
            h, A, B = res
    return out


# speedup vs baseline: 1.4808x; 1.0853x over previous
import functools

import jax
import jax.numpy as jnp
from jax import lax
from jax.experimental import pallas as pl
from jax.experimental.pallas import tpu as pltpu
from jax.experimental.pallas import tpu_sc as plsc

N = 4096
DIM = 3
H = 64
K = 32
OUT = 3

RB = 256
NBLK_TOPK = N // RB
NB = 512
NBLK_LAYER = N // NB
E = N * K


def _topk_body(xb_ref, xt_ref, we_ref, be_ref, wt_ref, wm_ref,
               dn_ref, idx_ref, h_ref, a_ref, b_ref, d_scr):
    blk = pl.program_id(0)
    xb = xb_ref[...]
    xt = xt_ref[...]
    d0 = xb[:, 0:1] - xt[0:1, :]
    d1 = xb[:, 1:2] - xt[1:2, :]
    d2 = xb[:, 2:3] - xt[2:3, :]
    dsq = d0 * d0 + d1 * d1 + d2 * d2

    rows = blk * RB + lax.broadcasted_iota(jnp.int32, (RB, N), 0)
    cols = lax.broadcasted_iota(jnp.int32, (RB, N), 1)
    d_scr[...] = jnp.where(rows == cols, jnp.inf, dsq)

    lane = lax.broadcasted_iota(jnp.int32, (RB, K), 1)

    def step(t, carry):
        dn_acc, idx_acc = carry
        d = d_scr[...]
        m = jnp.min(d, axis=1)
        idxi = jnp.min(jnp.where(d <= m[:, None], cols, N), axis=1)
        d_scr[...] = jnp.where(cols == idxi[:, None], jnp.inf, d)
        dn_acc = jnp.where(lane == t,
                           jnp.sqrt(jnp.maximum(m, 0.0))[:, None], dn_acc)
        idx_acc = jnp.where(lane == t, idxi[:, None], idx_acc)
        return dn_acc, idx_acc

    dn_acc, idx_acc = lax.fori_loop(
        0, K, step,
        (jnp.zeros((RB, K), jnp.float32), jnp.zeros((RB, K), jnp.int32)))
    dn_ref[...] = dn_acc
    idx_ref[...] = idx_acc

    h = (xb[:, 0:1] * we_ref[0:1, :] + xb[:, 1:2] * we_ref[1:2, :]
         + xb[:, 2:3] * we_ref[2:3, :] + be_ref[...])
    h_ref[...] = h
    a_ref[...] = jnp.dot(h, wt_ref[...], preferred_element_type=jnp.float32)
    b_ref[...] = jnp.dot(h, wm_ref[...], preferred_element_type=jnp.float32)


def _topk_call(x, xt, W_embed, b_embed2, wt0, wm0):
    full = lambda shape: pl.BlockSpec(shape, lambda i: (0, 0))
    return pl.pallas_call(
        _topk_body,
        grid=(NBLK_TOPK,),
        in_specs=[
            pl.BlockSpec((RB, DIM), lambda i: (i, 0)),
            full((DIM, N)),
            full((DIM, H)),
            full((1, H)),
            full((H, H)),
            full((H, _GW)),
        ],
        out_specs=[
            pl.BlockSpec((RB, K), lambda i: (i, 0)),
            pl.BlockSpec((RB, K), lambda i: (i, 0)),
            pl.BlockSpec((RB, H), lambda i: (i, 0)),
            pl.BlockSpec((RB, H), lambda i: (i, 0)),
            pl.BlockSpec((RB, _GW), lambda i: (i, 0)),
        ],
        out_shape=[
            jax.ShapeDtypeStruct((N, K), jnp.float32),
            jax.ShapeDtypeStruct((N, K), jnp.int32),
            jax.ShapeDtypeStruct((N, H), jnp.float32),
            jax.ShapeDtypeStruct((N, H), jnp.float32),
            jax.ShapeDtypeStruct((N, _GW), jnp.float32),
        ],
        scratch_shapes=[pltpu.VMEM((RB, N), jnp.float32)],
        compiler_params=pltpu.CompilerParams(
            dimension_semantics=("parallel",)),
    )(x, xt, W_embed, b_embed2, wt0, wm0)


_SC_NW = 32
_PER_W = E // _SC_NW
_CHUNK = 256
_NCH = _PER_W // _CHUNK
_SUB = _CHUNK // 128
_IDXR = _PER_W // 128
_GW = 2 * H


def _sc_gather_body(table_hbm, idx_hbm, out_hbm, idx_v, rows_v, tbl_sh,
                    gsem0, gsem1, osem0, osem1):
    sid = lax.axis_index("s")
    wid = sid * 2 + lax.axis_index("c")
    gsem = (gsem0, gsem1)
    osem = (osem0, osem1)

    @pl.when(sid == 0)
    def _stage_table():
        pltpu.sync_copy(table_hbm, tbl_sh)

    pltpu.sync_copy(
        idx_hbm.at[pl.ds(pl.multiple_of(wid * _IDXR, _IDXR), _IDXR)], idx_v)
    plsc.subcore_barrier()
    outc = [None] * _NCH
    for ch in range(_NCH):
        p = ch & 1
        if ch >= 2:
            outc[ch - 2].wait()
        gc = []
        for j in range(_SUB):
            gc.append(pltpu.async_copy(
                tbl_sh.at[idx_v.at[ch * _SUB + j]],
                rows_v.at[p, pl.ds(j * 128, 128)], gsem[p]))
        for c in gc:
            c.wait()
        row0 = pl.multiple_of(wid * _PER_W + ch * _CHUNK, _CHUNK)
        outc[ch] = pltpu.async_copy(
            rows_v.at[p], out_hbm.at[pl.ds(row0, _CHUNK)], osem[p])
    outc[_NCH - 2].wait()
    outc[_NCH - 1].wait()


@functools.lru_cache(maxsize=1)
def _sc_gather():
    return functools.partial(
        pl.kernel,
        out_type=jax.ShapeDtypeStruct((E, _GW), jnp.float32),
        mesh=plsc.VectorSubcoreMesh(core_axis_name="c", subcore_axis_name="s",
                                    num_cores=2, num_subcores=16),
        scratch_types=[
            pltpu.VMEM((_IDXR, 128), jnp.int32),
            pltpu.VMEM((2, _CHUNK, _GW), jnp.float32),
            pltpu.VMEM_SHARED((N, _GW), jnp.float32),
            pltpu.SemaphoreType.DMA,
            pltpu.SemaphoreType.DMA,
            pltpu.SemaphoreType.DMA,
            pltpu.SemaphoreType.DMA,
        ],
    )(_sc_gather_body)


def _silu(x):
    return x * (1.0 / (1.0 + jnp.exp(-x)))


def _layer_body(is_last,
                h_ref, a_ref, g_ref, dn_ref, wd_ref, bmi_ref, wmo_ref, bmo_ref,
                wuh_ref, wua_ref, bui_ref, wuo_ref, buo_ref, wn1_ref, wn2_ref,
                o1_ref, o2_ref, o3_ref):
    h = h_ref[...]
    g = g_ref[:, :H].reshape(NB, K, H)
    pre = (g + a_ref[...][:, None, :]
           + dn_ref[...][:, :, None] * wd_ref[...][None, :, :]
           + bmi_ref[...][None, :, :])
    s = jnp.sum(_silu(pre), axis=1)
    agg = (jnp.dot(s, wmo_ref[...], preferred_element_type=jnp.float32)
           + float(K) * bmo_ref[...])
    ui = (jnp.dot(h, wuh_ref[...], preferred_element_type=jnp.float32)
          + jnp.dot(agg, wua_ref[...], preferred_element_type=jnp.float32)
          + bui_ref[...])
    upd = (jnp.dot(_silu(ui), wuo_ref[...], preferred_element_type=jnp.float32)
           + buo_ref[...])
    hn = h + upd
    if is_last:
        o1_ref[...] = (jnp.dot(hn, wn1_ref[...],
                               preferred_element_type=jnp.float32)
                       + wn2_ref[...])
    else:
        o1_ref[...] = hn
        o2_ref[...] = jnp.dot(hn, wn1_ref[...],
                              preferred_element_type=jnp.float32)
        o3_ref[...] = jnp.dot(hn, wn2_ref[...],
                              preferred_element_type=jnp.float32)


def _layer_call(is_last, h, A, G, dn, wd, bmi, wmo, bmo,
                wuh, wua, bui, wuo, buo, wn1, wn2):
    full = lambda shape: pl.BlockSpec(shape, lambda i: (0, 0))
    row = lambda w: pl.BlockSpec((NB, w), lambda i: (i, 0))
    if is_last:
        out_specs = [pl.BlockSpec((NB, OUT), lambda i: (i, 0))]
        out_shape = [jax.ShapeDtypeStruct((N, OUT), jnp.float32)]
        body = functools.partial(_layer_body, True)

        def wrapped(*refs):
            body(*refs, None, None)
    else:
        out_specs = [row(H), row(H), row(_GW)]
        out_shape = [jax.ShapeDtypeStruct((N, H), jnp.float32)] * 2 + [
            jax.ShapeDtypeStruct((N, _GW), jnp.float32)]
        wrapped = functools.partial(_layer_body, False)
    res = pl.pallas_call(
        wrapped,
        grid=(NBLK_LAYER,),
        in_specs=[
            row(H), row(H),
            pl.BlockSpec((NB * K, _GW), lambda i: (i, 0)),
            row(K),
            full((1, H)), full((1, H)), full((H, H)), full((1, H)),
            full((H, H)), full((H, H)), full((1, H)), full((H, H)),
            full((1, H)),
            full(wn1.shape), full(wn2.shape),
        ],
        out_specs=out_specs,
        out_shape=out_shape,
        compiler_params=pltpu.CompilerParams(
            dimension_semantics=("parallel",)),
    )(h, A, G, dn, wd, bmi, wmo, bmo, wuh, wua, bui, wuo, buo, wn1, wn2)
    return res


def kernel(x, W_embed, b_embed,
           W_msg_in_0, b_msg_in_0, W_msg_out_0, b_msg_out_0,
           W_upd_in_0, b_upd_in_0, W_upd_out_0, b_upd_out_0,
           W_msg_in_1, b_msg_in_1, W_msg_out_1, b_msg_out_1,
           W_upd_in_1, b_upd_in_1, W_upd_out_1, b_upd_out_1,
           W_msg_in_2, b_msg_in_2, W_msg_out_2, b_msg_out_2,
           W_upd_in_2, b_upd_in_2, W_upd_out_2, b_upd_out_2,
           W_out, b_out):
    msg_in = [W_msg_in_0, W_msg_in_1, W_msg_in_2]
    b_msg_in = [b_msg_in_0, b_msg_in_1, b_msg_in_2]
    msg_out = [W_msg_out_0, W_msg_out_1, W_msg_out_2]
    b_msg_out = [b_msg_out_0, b_msg_out_1, b_msg_out_2]
    upd_in = [W_upd_in_0, W_upd_in_1, W_upd_in_2]
    b_upd_in = [b_upd_in_0, b_upd_in_1, b_upd_in_2]
    upd_out = [W_upd_out_0, W_upd_out_1, W_upd_out_2]
    b_upd_out = [b_upd_out_0, b_upd_out_1, b_upd_out_2]

    r2 = lambda v: v.reshape(1, -1)
    padw = lambda w: jnp.pad(w, ((0, 0), (0, _GW - w.shape[1])))
    xt = x.T
    dn, idx, h, A, B = _topk_call(
        x, xt, W_embed, r2(b_embed), W_msg_in_0[:H],
        padw(W_msg_in_0[H:2 * H]))
    idx2d = idx.reshape(E // 128, 128)

    out = None
    for l in range(3):
        G = _sc_gather()(B, idx2d)
        is_last = l == 2
        if is_last:
            wn1, wn2 = W_out, r2(b_out)
        else:
            wn1, wn2 = msg_in[l + 1][:H], padw(msg_in[l + 1][H:2 * H])
        res = _layer_call(
            is_last, h, A, G, dn,
            r2(msg_in[l][2 * H]), r2(b_msg_in[l]), msg_out[l],
            r2(b_msg_out[l]),
            upd_in[l][:H], upd_in[l][H:], r2(b_upd_in[l]),
            upd_out[l], r2(b_upd_out[l]), wn1, wn2)
        if is_last:
            out = res[0]
        else:
            h, A, B = res
    return out


# speedup vs baseline: 1.5273x; 1.0314x over previous
import functools

import jax
import jax.numpy as jnp
from jax import lax
from jax.experimental import pallas as pl
from jax.experimental.pallas import tpu as pltpu
from jax.experimental.pallas import tpu_sc as plsc

N = 4096
DIM = 3
H = 64
K = 32
OUT = 3

RB = 512
NBLK_TOPK = N // RB
NB = 512
NBLK_LAYER = N // NB
E = N * K


def _topk_body(xb_ref, xt_ref, we_ref, be_ref, wt_ref, wm_ref,
               dn_ref, idx_ref, h_ref, a_ref, b_ref, d_scr):
    blk = pl.program_id(0)
    xb = xb_ref[...]
    xt = xt_ref[...]
    d0 = xb[:, 0:1] - xt[0:1, :]
    d1 = xb[:, 1:2] - xt[1:2, :]
    d2 = xb[:, 2:3] - xt[2:3, :]
    dsq = d0 * d0 + d1 * d1 + d2 * d2

    rows = blk * RB + lax.broadcasted_iota(jnp.int32, (RB, N), 0)
    cols = lax.broadcasted_iota(jnp.int32, (RB, N), 1)
    d_scr[...] = jnp.where(rows == cols, jnp.inf, dsq)

    lane = lax.broadcasted_iota(jnp.int32, (RB, K), 1)

    def step(t, carry):
        dn_acc, idx_acc = carry
        d = d_scr[...]
        m = jnp.min(d, axis=1)
        idxi = jnp.min(jnp.where(d <= m[:, None], cols, N), axis=1)
        d_scr[...] = jnp.where(cols == idxi[:, None], jnp.inf, d)
        dn_acc = jnp.where(lane == t,
                           jnp.sqrt(jnp.maximum(m, 0.0))[:, None], dn_acc)
        idx_acc = jnp.where(lane == t, idxi[:, None], idx_acc)
        return dn_acc, idx_acc

    dn_acc, idx_acc = lax.fori_loop(
        0, K, step,
        (jnp.zeros((RB, K), jnp.float32), jnp.zeros((RB, K), jnp.int32)))
    dn_ref[...] = dn_acc
    idx_ref[...] = idx_acc

    h = (xb[:, 0:1] * we_ref[0:1, :] + xb[:, 1:2] * we_ref[1:2, :]
         + xb[:, 2:3] * we_ref[2:3, :] + be_ref[...])
    h_ref[...] = h
    a_ref[...] = jnp.dot(h, wt_ref[...], preferred_element_type=jnp.float32)
    b_ref[...] = jnp.dot(h, wm_ref[...], preferred_element_type=jnp.float32)


def _topk_call(x, xt, W_embed, b_embed2, wt0, wm0):
    full = lambda shape: pl.BlockSpec(shape, lambda i: (0, 0))
    return pl.pallas_call(
        _topk_body,
        grid=(NBLK_TOPK,),
        in_specs=[
            pl.BlockSpec((RB, DIM), lambda i: (i, 0)),
            full((DIM, N)),
            full((DIM, H)),
            full((1, H)),
            full((H, H)),
            full((H, _GW)),
        ],
        out_specs=[
            pl.BlockSpec((RB, K), lambda i: (i, 0)),
            pl.BlockSpec((RB, K), lambda i: (i, 0)),
            pl.BlockSpec((RB, H), lambda i: (i, 0)),
            pl.BlockSpec((RB, H), lambda i: (i, 0)),
            pl.BlockSpec((RB, _GW), lambda i: (i, 0)),
        ],
        out_shape=[
            jax.ShapeDtypeStruct((N, K), jnp.float32),
            jax.ShapeDtypeStruct((N, K), jnp.int32),
            jax.ShapeDtypeStruct((N, H), jnp.float32),
            jax.ShapeDtypeStruct((N, H), jnp.float32),
            jax.ShapeDtypeStruct((N, _GW), jnp.float32),
        ],
        scratch_shapes=[pltpu.VMEM((RB, N), jnp.float32)],
        compiler_params=pltpu.CompilerParams(
            dimension_semantics=("parallel",)),
    )(x, xt, W_embed, b_embed2, wt0, wm0)


_SC_NW = 32
_PER_W = E // _SC_NW
_CHUNK = 256
_NCH = _PER_W // _CHUNK
_SUB = _CHUNK // 128
_IDXR = _PER_W // 128
_GW = 2 * H


def _sc_gather_body(table_hbm, idx_hbm, out_hbm, idx_v, rows_v, tbl_sh,
                    gsem0, gsem1, osem0, osem1):
    sid = lax.axis_index("s")
    wid = sid * 2 + lax.axis_index("c")
    gsem = (gsem0, gsem1)
    osem = (osem0, osem1)

    @pl.when(sid == 0)
    def _stage_table():
        pltpu.sync_copy(table_hbm, tbl_sh)

    pltpu.sync_copy(
        idx_hbm.at[pl.ds(pl.multiple_of(wid * _IDXR, _IDXR), _IDXR)], idx_v)
    plsc.subcore_barrier()
    outc = [None] * _NCH
    for ch in range(_NCH):
        p = ch & 1
        if ch >= 2:
            outc[ch - 2].wait()
        gc = []
        for j in range(_SUB):
            gc.append(pltpu.async_copy(
                tbl_sh.at[idx_v.at[ch * _SUB + j]],
                rows_v.at[p, pl.ds(j * 128, 128)], gsem[p]))
        for c in gc:
            c.wait()
        row0 = pl.multiple_of(wid * _PER_W + ch * _CHUNK, _CHUNK)
        outc[ch] = pltpu.async_copy(
            rows_v.at[p], out_hbm.at[pl.ds(row0, _CHUNK)], osem[p])
    outc[_NCH - 2].wait()
    outc[_NCH - 1].wait()


@functools.lru_cache(maxsize=1)
def _sc_gather():
    return functools.partial(
        pl.kernel,
        out_type=jax.ShapeDtypeStruct((E, _GW), jnp.float32),
        mesh=plsc.VectorSubcoreMesh(core_axis_name="c", subcore_axis_name="s",
                                    num_cores=2, num_subcores=16),
        scratch_types=[
            pltpu.VMEM((_IDXR, 128), jnp.int32),
            pltpu.VMEM((2, _CHUNK, _GW), jnp.float32),
            pltpu.VMEM_SHARED((N, _GW), jnp.float32),
            pltpu.SemaphoreType.DMA,
            pltpu.SemaphoreType.DMA,
            pltpu.SemaphoreType.DMA,
            pltpu.SemaphoreType.DMA,
        ],
    )(_sc_gather_body)


def _silu(x):
    return x * (1.0 / (1.0 + jnp.exp(-x)))


def _layer_body(is_last,
                h_ref, a_ref, g_ref, dn_ref, wd_ref, bmi_ref, wmo_ref, bmo_ref,
                wuh_ref, wua_ref, bui_ref, wuo_ref, buo_ref, wn1_ref, wn2_ref,
                o1_ref, o2_ref, o3_ref):
    h = h_ref[...]
    g = g_ref[:, :H].reshape(NB, K, H)
    pre = (g + a_ref[...][:, None, :]
           + dn_ref[...][:, :, None] * wd_ref[...][None, :, :]
           + bmi_ref[...][None, :, :])
    s = jnp.sum(_silu(pre), axis=1)
    agg = (jnp.dot(s, wmo_ref[...], preferred_element_type=jnp.float32)
           + float(K) * bmo_ref[...])
    ui = (jnp.dot(h, wuh_ref[...], preferred_element_type=jnp.float32)
          + jnp.dot(agg, wua_ref[...], preferred_element_type=jnp.float32)
          + bui_ref[...])
    upd = (jnp.dot(_silu(ui), wuo_ref[...], preferred_element_type=jnp.float32)
           + buo_ref[...])
    hn = h + upd
    if is_last:
        o1_ref[...] = (jnp.dot(hn, wn1_ref[...],
                               preferred_element_type=jnp.float32)
                       + wn2_ref[...])
    else:
        o1_ref[...] = hn
        o2_ref[...] = jnp.dot(hn, wn1_ref[...],
                              preferred_element_type=jnp.float32)
        o3_ref[...] = jnp.dot(hn, wn2_ref[...],
                              preferred_element_type=jnp.float32)


def _layer_call(is_last, h, A, G, dn, wd, bmi, wmo, bmo,
                wuh, wua, bui, wuo, buo, wn1, wn2):
    full = lambda shape: pl.BlockSpec(shape, lambda i: (0, 0))
    row = lambda w: pl.BlockSpec((NB, w), lambda i: (i, 0))
    if is_last:
        out_specs = [pl.BlockSpec((NB, OUT), lambda i: (i, 0))]
        out_shape = [jax.ShapeDtypeStruct((N, OUT), jnp.float32)]
        body = functools.partial(_layer_body, True)

        def wrapped(*refs):
            body(*refs, None, None)
    else:
        out_specs = [row(H), row(H), row(_GW)]
        out_shape = [jax.ShapeDtypeStruct((N, H), jnp.float32)] * 2 + [
            jax.ShapeDtypeStruct((N, _GW), jnp.float32)]
        wrapped = functools.partial(_layer_body, False)
    res = pl.pallas_call(
        wrapped,
        grid=(NBLK_LAYER,),
        in_specs=[
            row(H), row(H),
            pl.BlockSpec((NB * K, _GW), lambda i: (i, 0)),
            row(K),
            full((1, H)), full((1, H)), full((H, H)), full((1, H)),
            full((H, H)), full((H, H)), full((1, H)), full((H, H)),
            full((1, H)),
            full(wn1.shape), full(wn2.shape),
        ],
        out_specs=out_specs,
        out_shape=out_shape,
        compiler_params=pltpu.CompilerParams(
            dimension_semantics=("parallel",)),
    )(h, A, G, dn, wd, bmi, wmo, bmo, wuh, wua, bui, wuo, buo, wn1, wn2)
    return res


def kernel(x, W_embed, b_embed,
           W_msg_in_0, b_msg_in_0, W_msg_out_0, b_msg_out_0,
           W_upd_in_0, b_upd_in_0, W_upd_out_0, b_upd_out_0,
           W_msg_in_1, b_msg_in_1, W_msg_out_1, b_msg_out_1,
           W_upd_in_1, b_upd_in_1, W_upd_out_1, b_upd_out_1,
           W_msg_in_2, b_msg_in_2, W_msg_out_2, b_msg_out_2,
           W_upd_in_2, b_upd_in_2, W_upd_out_2, b_upd_out_2,
           W_out, b_out):
    msg_in = [W_msg_in_0, W_msg_in_1, W_msg_in_2]
    b_msg_in = [b_msg_in_0, b_msg_in_1, b_msg_in_2]
    msg_out = [W_msg_out_0, W_msg_out_1, W_msg_out_2]
    b_msg_out = [b_msg_out_0, b_msg_out_1, b_msg_out_2]
    upd_in = [W_upd_in_0, W_upd_in_1, W_upd_in_2]
    b_upd_in = [b_upd_in_0, b_upd_in_1, b_upd_in_2]
    upd_out = [W_upd_out_0, W_upd_out_1, W_upd_out_2]
    b_upd_out = [b_upd_out_0, b_upd_out_1, b_upd_out_2]

    r2 = lambda v: v.reshape(1, -1)
    padw = lambda w: jnp.pad(w, ((0, 0), (0, _GW - w.shape[1])))
    xt = x.T
    dn, idx, h, A, B = _topk_call(
        x, xt, W_embed, r2(b_embed), W_msg_in_0[:H],
        padw(W_msg_in_0[H:2 * H]))
    idx2d = idx.reshape(E // 128, 128)

    out = None
    for l in range(3):
        G = _sc_gather()(B, idx2d)
        is_last = l == 2
        if is_last:
            wn1, wn2 = W_out, r2(b_out)
        else:
            wn1, wn2 = msg_in[l + 1][:H], padw(msg_in[l + 1][H:2 * H])
        res = _layer_call(
            is_last, h, A, G, dn,
            r2(msg_in[l][2 * H]), r2(b_msg_in[l]), msg_out[l],
            r2(b_msg_out[l]),
            upd_in[l][:H], upd_in[l][H:], r2(b_upd_in[l]),
            upd_out[l], r2(b_upd_out[l]), wn1, wn2)
        if is_last:
            out = res[0]
        else:
            h, A, B = res
    return out


# speedup vs baseline: 1.5529x; 1.0167x over previous
import functools

import jax
import jax.numpy as jnp
from jax import lax
from jax.experimental import pallas as pl
from jax.experimental.pallas import tpu as pltpu
from jax.experimental.pallas import tpu_sc as plsc

N = 4096
DIM = 3
H = 64
K = 32
OUT = 3

RB = 1024
NBLK_TOPK = N // RB
NB = 1024
NBLK_LAYER = N // NB
E = N * K


def _topk_body(xb_ref, xt_ref, we_ref, be_ref, wt_ref, wm_ref,
               dn_ref, idx_ref, h_ref, a_ref, b_ref, d_scr):
    blk = pl.program_id(0)
    xb = xb_ref[...]
    xt = xt_ref[...]
    d0 = xb[:, 0:1] - xt[0:1, :]
    d1 = xb[:, 1:2] - xt[1:2, :]
    d2 = xb[:, 2:3] - xt[2:3, :]
    dsq = d0 * d0 + d1 * d1 + d2 * d2

    rows = blk * RB + lax.broadcasted_iota(jnp.int32, (RB, N), 0)
    cols = lax.broadcasted_iota(jnp.int32, (RB, N), 1)
    d_scr[...] = jnp.where(rows == cols, jnp.inf, dsq)

    lane = lax.broadcasted_iota(jnp.int32, (RB, K), 1)

    def step(t, carry):
        dn_acc, idx_acc = carry
        d = d_scr[...]
        m = jnp.min(d, axis=1)
        idxi = jnp.min(jnp.where(d <= m[:, None], cols, N), axis=1)
        d_scr[...] = jnp.where(cols == idxi[:, None], jnp.inf, d)
        dn_acc = jnp.where(lane == t,
                           jnp.sqrt(jnp.maximum(m, 0.0))[:, None], dn_acc)
        idx_acc = jnp.where(lane == t, idxi[:, None], idx_acc)
        return dn_acc, idx_acc

    dn_acc, idx_acc = lax.fori_loop(
        0, K, step,
        (jnp.zeros((RB, K), jnp.float32), jnp.zeros((RB, K), jnp.int32)))
    dn_ref[...] = dn_acc
    idx_ref[...] = idx_acc

    h = (xb[:, 0:1] * we_ref[0:1, :] + xb[:, 1:2] * we_ref[1:2, :]
         + xb[:, 2:3] * we_ref[2:3, :] + be_ref[...])
    h_ref[...] = h
    a_ref[...] = jnp.dot(h, wt_ref[...], preferred_element_type=jnp.float32)
    b_ref[...] = jnp.dot(h, wm_ref[...], preferred_element_type=jnp.float32)


def _topk_call(x, xt, W_embed, b_embed2, wt0, wm0):
    full = lambda shape: pl.BlockSpec(shape, lambda i: (0, 0))
    return pl.pallas_call(
        _topk_body,
        grid=(NBLK_TOPK,),
        in_specs=[
            pl.BlockSpec((RB, DIM), lambda i: (i, 0)),
            full((DIM, N)),
            full((DIM, H)),
            full((1, H)),
            full((H, H)),
            full((H, _GW)),
        ],
        out_specs=[
            pl.BlockSpec((RB, K), lambda i: (i, 0)),
            pl.BlockSpec((RB, K), lambda i: (i, 0)),
            pl.BlockSpec((RB, H), lambda i: (i, 0)),
            pl.BlockSpec((RB, H), lambda i: (i, 0)),
            pl.BlockSpec((RB, _GW), lambda i: (i, 0)),
        ],
        out_shape=[
            jax.ShapeDtypeStruct((N, K), jnp.float32),
            jax.ShapeDtypeStruct((N, K), jnp.int32),
            jax.ShapeDtypeStruct((N, H), jnp.float32),
            jax.ShapeDtypeStruct((N, H), jnp.float32),
            jax.ShapeDtypeStruct((N, _GW), jnp.float32),
        ],
        scratch_shapes=[pltpu.VMEM((RB, N), jnp.float32)],
        compiler_params=pltpu.CompilerParams(
            dimension_semantics=("parallel",)),
    )(x, xt, W_embed, b_embed2, wt0, wm0)


_SC_NW = 32
_PER_W = E // _SC_NW
_CHUNK = 256
_NCH = _PER_W // _CHUNK
_SUB = _CHUNK // 128
_IDXR = _PER_W // 128
_GW = 2 * H


def _sc_gather_body(table_hbm, idx_hbm, out_hbm, idx_v, rows_v, tbl_sh,
                    gsem0, gsem1, osem0, osem1):
    sid = lax.axis_index("s")
    wid = sid * 2 + lax.axis_index("c")
    gsem = (gsem0, gsem1)
    osem = (osem0, osem1)

    @pl.when(sid == 0)
    def _stage_table():
        pltpu.sync_copy(table_hbm, tbl_sh)

    pltpu.sync_copy(
        idx_hbm.at[pl.ds(pl.multiple_of(wid * _IDXR, _IDXR), _IDXR)], idx_v)
    plsc.subcore_barrier()
    outc = [None] * _NCH
    for ch in range(_NCH):
        p = ch & 1
        if ch >= 2:
            outc[ch - 2].wait()
        gc = []
        for j in range(_SUB):
            gc.append(pltpu.async_copy(
                tbl_sh.at[idx_v.at[ch * _SUB + j]],
                rows_v.at[p, pl.ds(j * 128, 128)], gsem[p]))
        for c in gc:
            c.wait()
        row0 = pl.multiple_of(wid * _PER_W + ch * _CHUNK, _CHUNK)
        outc[ch] = pltpu.async_copy(
            rows_v.at[p], out_hbm.at[pl.ds(row0, _CHUNK)], osem[p])
    outc[_NCH - 2].wait()
    outc[_NCH - 1].wait()


@functools.lru_cache(maxsize=1)
def _sc_gather():
    return functools.partial(
        pl.kernel,
        out_type=jax.ShapeDtypeStruct((E, _GW), jnp.float32),
        mesh=plsc.VectorSubcoreMesh(core_axis_name="c", subcore_axis_name="s",
                                    num_cores=2, num_subcores=16),
        scratch_types=[
            pltpu.VMEM((_IDXR, 128), jnp.int32),
            pltpu.VMEM((2, _CHUNK, _GW), jnp.float32),
            pltpu.VMEM_SHARED((N, _GW), jnp.float32),
            pltpu.SemaphoreType.DMA,
            pltpu.SemaphoreType.DMA,
            pltpu.SemaphoreType.DMA,
            pltpu.SemaphoreType.DMA,
        ],
    )(_sc_gather_body)


def _silu(x):
    return x * (1.0 / (1.0 + jnp.exp(-x)))


def _layer_body(is_last,
                h_ref, a_ref, g_ref, dn_ref, wd_ref, bmi_ref, wmo_ref, bmo_ref,
                wuh_ref, wua_ref, bui_ref, wuo_ref, buo_ref, wn1_ref, wn2_ref,
                o1_ref, o2_ref, o3_ref):
    h = h_ref[...]
    g = g_ref[:, :H].reshape(NB, K, H)
    pre = (g + a_ref[...][:, None, :]
           + dn_ref[...][:, :, None] * wd_ref[...][None, :, :]
           + bmi_ref[...][None, :, :])
    s = jnp.sum(_silu(pre), axis=1)
    agg = (jnp.dot(s, wmo_ref[...], preferred_element_type=jnp.float32)
           + float(K) * bmo_ref[...])
    ui = (jnp.dot(h, wuh_ref[...], preferred_element_type=jnp.float32)
          + jnp.dot(agg, wua_ref[...], preferred_element_type=jnp.float32)
          + bui_ref[...])
    upd = (jnp.dot(_silu(ui), wuo_ref[...], preferred_element_type=jnp.float32)
           + buo_ref[...])
    hn = h + upd
    if is_last:
        o1_ref[...] = (jnp.dot(hn, wn1_ref[...],
                               preferred_element_type=jnp.float32)
                       + wn2_ref[...])
    else:
        o1_ref[...] = hn
        o2_ref[...] = jnp.dot(hn, wn1_ref[...],
                              preferred_element_type=jnp.float32)
        o3_ref[...] = jnp.dot(hn, wn2_ref[...],
                              preferred_element_type=jnp.float32)


def _layer_call(is_last, h, A, G, dn, wd, bmi, wmo, bmo,
                wuh, wua, bui, wuo, buo, wn1, wn2):
    full = lambda shape: pl.BlockSpec(shape, lambda i: (0, 0))
    row = lambda w: pl.BlockSpec((NB, w), lambda i: (i, 0))
    if is_last:
        out_specs = [pl.BlockSpec((NB, OUT), lambda i: (i, 0))]
        out_shape = [jax.ShapeDtypeStruct((N, OUT), jnp.float32)]
        body = functools.partial(_layer_body, True)

        def wrapped(*refs):
            body(*refs, None, None)
    else:
        out_specs = [row(H), row(H), row(_GW)]
        out_shape = [jax.ShapeDtypeStruct((N, H), jnp.float32)] * 2 + [
            jax.ShapeDtypeStruct((N, _GW), jnp.float32)]
        wrapped = functools.partial(_layer_body, False)
    res = pl.pallas_call(
        wrapped,
        grid=(NBLK_LAYER,),
        in_specs=[
            row(H), row(H),
            pl.BlockSpec((NB * K, _GW), lambda i: (i, 0)),
            row(K),
            full((1, H)), full((1, H)), full((H, H)), full((1, H)),
            full((H, H)), full((H, H)), full((1, H)), full((H, H)),
            full((1, H)),
            full(wn1.shape), full(wn2.shape),
        ],
        out_specs=out_specs,
        out_shape=out_shape,
        compiler_params=pltpu.CompilerParams(
            dimension_semantics=("parallel",)),
    )(h, A, G, dn, wd, bmi, wmo, bmo, wuh, wua, bui, wuo, buo, wn1, wn2)
    return res


def kernel(x, W_embed, b_embed,
           W_msg_in_0, b_msg_in_0, W_msg_out_0, b_msg_out_0,
           W_upd_in_0, b_upd_in_0, W_upd_out_0, b_upd_out_0,
           W_msg_in_1, b_msg_in_1, W_msg_out_1, b_msg_out_1,
           W_upd_in_1, b_upd_in_1, W_upd_out_1, b_upd_out_1,
           W_msg_in_2, b_msg_in_2, W_msg_out_2, b_msg_out_2,
           W_upd_in_2, b_upd_in_2, W_upd_out_2, b_upd_out_2,
           W_out, b_out):
    msg_in = [W_msg_in_0, W_msg_in_1, W_msg_in_2]
    b_msg_in = [b_msg_in_0, b_msg_in_1, b_msg_in_2]
    msg_out = [W_msg_out_0, W_msg_out_1, W_msg_out_2]
    b_msg_out = [b_msg_out_0, b_msg_out_1, b_msg_out_2]
    upd_in = [W_upd_in_0, W_upd_in_1, W_upd_in_2]
    b_upd_in = [b_upd_in_0, b_upd_in_1, b_upd_in_2]
    upd_out = [W_upd_out_0, W_upd_out_1, W_upd_out_2]
    b_upd_out = [b_upd_out_0, b_upd_out_1, b_upd_out_2]

    r2 = lambda v: v.reshape(1, -1)
    padw = lambda w: jnp.pad(w, ((0, 0), (0, _GW - w.shape[1])))
    xt = x.T
    dn, idx, h, A, B = _topk_call(
        x, xt, W_embed, r2(b_embed), W_msg_in_0[:H],
        padw(W_msg_in_0[H:2 * H]))
    idx2d = idx.reshape(E // 128, 128)

    out = None
    for l in range(3):
        G = _sc_gather()(B, idx2d)
        is_last = l == 2
        if is_last:
            wn1, wn2 = W_out, r2(b_out)
        else:
            wn1, wn2 = msg_in[l + 1][:H], padw(msg_in[l + 1][H:2 * H])
        res = _layer_call(
            is_last, h, A, G, dn,
            r2(msg_in[l][2 * H]), r2(b_msg_in[l]), msg_out[l],
            r2(b_msg_out[l]),
            upd_in[l][:H], upd_in[l][H:], r2(b_upd_in[l]),
            upd_out[l], r2(b_upd_out[l]), wn1, wn2)
        if is_last:
            out = res[0]
        else:
            h, A, B = res
    return out
